# baseline (device time: 225418 ns/iter reference)
import jax
import jax.numpy as jnp
from jax import lax
from jax.experimental import pallas as pl
from jax.experimental.pallas import tpu as pltpu

N_DEV = 32
M = 1024
N = 1024
CHUNK = M // N_DEV


def kernel(x, W1, W2):
    x = x.astype(jnp.bfloat16)
    W1 = W1.astype(jnp.bfloat16)
    W2 = W2.astype(jnp.bfloat16)

    def body(x_ref, w1_ref, w2_ref, out_ref, recv_rs, send_sem, rs_sems, ag_sems):
        my = lax.axis_index("i")
        left = lax.rem(my + N_DEV - 1, N_DEV)
        right = lax.rem(my + 1, N_DEV)

        h = jnp.dot(x_ref[...], w1_ref[...], preferred_element_type=jnp.float32)
        h = jnp.maximum(h, 0.0).astype(jnp.bfloat16)
        out_ref[...] = jnp.dot(h, w2_ref[...], preferred_element_type=jnp.float32)

        barrier = pltpu.get_barrier_semaphore()
        for nbr in (left, right):
            pl.semaphore_signal(
                barrier, inc=1,
                device_id=(nbr,), device_id_type=pl.DeviceIdType.MESH,
            )
        pl.semaphore_wait(barrier, 2)

        for hop in range(N_DEV - 1):
            c = lax.rem(my - hop + N_DEV, N_DEV)
            rdma = pltpu.make_async_remote_copy(
                src_ref=out_ref.at[pl.ds(c * CHUNK, CHUNK), :],
                dst_ref=recv_rs.at[hop],
                send_sem=send_sem,
                recv_sem=rs_sems.at[hop],
                device_id=(right,),
                device_id_type=pl.DeviceIdType.MESH,
            )
            rdma.start()
            rdma.wait()
            d = lax.rem(my - hop - 1 + N_DEV, N_DEV)
            row = pl.ds(d * CHUNK, CHUNK)
            out_ref[row, :] = out_ref[row, :] + recv_rs[hop, :, :]

        for hop in range(N_DEV - 1):
            c = lax.rem(my + 1 - hop + N_DEV, N_DEV)
            sl = pl.ds(c * CHUNK, CHUNK)
            rdma = pltpu.make_async_remote_copy(
                src_ref=out_ref.at[sl, :],
                dst_ref=out_ref.at[sl, :],
                send_sem=send_sem,
                recv_sem=ag_sems.at[hop],
                device_id=(right,),
                device_id_type=pl.DeviceIdType.MESH,
            )
            rdma.start()
            rdma.wait()

    return pl.pallas_call(
        body,
        out_shape=jax.ShapeDtypeStruct((M, N), jnp.float32),
        in_specs=[
            pl.BlockSpec(memory_space=pltpu.VMEM),
            pl.BlockSpec(memory_space=pltpu.VMEM),
            pl.BlockSpec(memory_space=pltpu.VMEM),
        ],
        out_specs=pl.BlockSpec(memory_space=pltpu.VMEM),
        scratch_shapes=[
            pltpu.VMEM((N_DEV - 1, CHUNK, N), jnp.float32),
            pltpu.SemaphoreType.DMA,
            pltpu.SemaphoreType.DMA((N_DEV - 1,)),
            pltpu.SemaphoreType.DMA((N_DEV - 1,)),
        ],
        compiler_params=pltpu.CompilerParams(collective_id=0),
    )(x, W1, W2)


# device time: 88126 ns/iter; 2.5579x vs baseline; 2.5579x over previous
import jax
import jax.numpy as jnp
from jax import lax
from jax.experimental import pallas as pl
from jax.experimental.pallas import tpu as pltpu

N_DEV = 32
M = 1024
N = 1024

RS_MASKS = (1, 8, 2, 4, 16)
RS_HALF = (512, 256, 128, 64, 32)
RS_OFF = (0, 512, 768, 896, 960)


def kernel(x, W1, W2):
    x = x.astype(jnp.bfloat16)
    W1 = W1.astype(jnp.bfloat16)
    W2 = W2.astype(jnp.bfloat16)

    def body(x_ref, w1_ref, w2_ref, out_ref, acc, stage, send_sem, rs_sems, ag_sems):
        my = lax.axis_index("i")

        h = jnp.dot(x_ref[...], w1_ref[...], preferred_element_type=jnp.float32)
        h = jnp.maximum(h, 0.0).astype(jnp.bfloat16)
        p = jnp.dot(h, w2_ref[...], preferred_element_type=jnp.float32)
        acc[...] = p.astype(jnp.bfloat16)

        barrier = pltpu.get_barrier_semaphore()
        for m in RS_MASKS:
            pl.semaphore_signal(
                barrier, inc=1,
                device_id=(my ^ m,), device_id_type=pl.DeviceIdType.MESH,
            )
        pl.semaphore_wait(barrier, len(RS_MASKS))

        lo = jnp.int32(0)
        for r, (m, half) in enumerate(zip(RS_MASKS, RS_HALF)):
            partner = my ^ m
            b = jnp.where((my & m) != 0, 1, 0).astype(jnp.int32)
            keep_lo = lo + b * half
            send_lo = lo + (1 - b) * half
            rdma = pltpu.make_async_remote_copy(
                src_ref=acc.at[pl.ds(send_lo, half), :],
                dst_ref=stage.at[pl.ds(RS_OFF[r], half), :],
                send_sem=send_sem,
                recv_sem=rs_sems.at[r],
                device_id=(partner,),
                device_id_type=pl.DeviceIdType.MESH,
            )
            rdma.start()
            rdma.wait()
            krows = pl.ds(keep_lo, half)
            acc[krows, :] = acc[krows, :] + stage[pl.ds(RS_OFF[r], half), :]
            lo = keep_lo

        sz = M // N_DEV
        for r, m in enumerate(reversed(RS_MASKS)):
            partner = my ^ m
            b = jnp.where((my & m) != 0, 1, 0).astype(jnp.int32)
            seg = pl.ds(lo, sz)
            rdma = pltpu.make_async_remote_copy(
                src_ref=acc.at[seg, :],
                dst_ref=acc.at[seg, :],
                send_sem=send_sem,
                recv_sem=ag_sems.at[r],
                device_id=(partner,),
                device_id_type=pl.DeviceIdType.MESH,
            )
            rdma.start()
            rdma.wait()
            lo = lo - b * sz
            sz = sz * 2

        out_ref[...] = acc[...].astype(jnp.float32)

    return pl.pallas_call(
        body,
        out_shape=jax.ShapeDtypeStruct((M, N), jnp.float32),
        in_specs=[
            pl.BlockSpec(memory_space=pltpu.VMEM),
            pl.BlockSpec(memory_space=pltpu.VMEM),
            pl.BlockSpec(memory_space=pltpu.VMEM),
        ],
        out_specs=pl.BlockSpec(memory_space=pltpu.VMEM),
        scratch_shapes=[
            pltpu.VMEM((M, N), jnp.bfloat16),
            pltpu.VMEM((M, N), jnp.bfloat16),
            pltpu.SemaphoreType.DMA,
            pltpu.SemaphoreType.DMA((5,)),
            pltpu.SemaphoreType.DMA((5,)),
        ],
        compiler_params=pltpu.CompilerParams(collective_id=0),
    )(x, W1, W2)


# device time: 75150 ns/iter; 2.9996x vs baseline; 1.1727x over previous
import jax
import jax.numpy as jnp
from jax import lax
from jax.experimental import pallas as pl
from jax.experimental.pallas import tpu as pltpu

N_DEV = 32
M = 1024
N = 1024
COL = N // 2

RS_MASKS = (1, 8, 2, 4, 16)
RS_HALF = (512, 256, 128, 64, 32)
RS_OFF = (0, 512, 768, 896, 960)
AG_MASKS = tuple(reversed(RS_MASKS))
AG_SZ = (32, 64, 128, 256, 512)


def kernel(x, W1, W2):
    x = x.astype(jnp.bfloat16)
    W1 = W1.astype(jnp.bfloat16)
    W2 = W2.astype(jnp.bfloat16)

    def body(x_ref, w1_ref, w2_ref, out_ref, acc, stage,
             send_a, send_b, rs_a, rs_b, ag_a, ag_b):
        my = lax.axis_index("i")
        send_sems = (send_a, send_b)
        rs_sems = (rs_a, rs_b)
        ag_sems = (ag_a, ag_b)
        col_off = (0, COL)

        barrier = pltpu.get_barrier_semaphore()
        for m in RS_MASKS:
            pl.semaphore_signal(
                barrier, inc=1,
                device_id=(my ^ m,), device_id_type=pl.DeviceIdType.MESH,
            )
        pl.semaphore_wait(barrier, len(RS_MASKS))

        bits = [jnp.where((my & m) != 0, 1, 0).astype(jnp.int32) for m in RS_MASKS]
        rs_lo = [jnp.int32(0)]
        rs_send_lo = []
        for r, half in enumerate(RS_HALF):
            rs_send_lo.append(rs_lo[r] + (1 - bits[r]) * half)
            rs_lo.append(rs_lo[r] + bits[r] * half)
        ag_lo = [rs_lo[-1]]
        for r, m in enumerate(AG_MASKS):
            b = bits[RS_MASKS.index(m)]
            ag_lo.append(ag_lo[r] - b * AG_SZ[r])

        def rs_desc(s, r):
            half = RS_HALF[r]
            cols = pl.ds(col_off[s], COL)
            return pltpu.make_async_remote_copy(
                src_ref=acc.at[pl.ds(rs_send_lo[r], half), cols],
                dst_ref=stage.at[pl.ds(RS_OFF[r], half), cols],
                send_sem=send_sems[s],
                recv_sem=rs_sems[s].at[r],
                device_id=(my ^ RS_MASKS[r],),
                device_id_type=pl.DeviceIdType.MESH,
            )

        def ag_desc(s, r):
            seg = acc.at[pl.ds(ag_lo[r], AG_SZ[r]), pl.ds(col_off[s], COL)]
            return pltpu.make_async_remote_copy(
                src_ref=seg, dst_ref=seg,
                send_sem=send_sems[s],
                recv_sem=ag_sems[s].at[r],
                device_id=(my ^ AG_MASKS[r],),
                device_id_type=pl.DeviceIdType.MESH,
            )

        h = jnp.dot(x_ref[...], w1_ref[...], preferred_element_type=jnp.float32)
        h = jnp.maximum(h, 0.0).astype(jnp.bfloat16)
        pa = jnp.dot(h, w2_ref[:, 0:COL], preferred_element_type=jnp.float32)
        acc[:, 0:COL] = pa.astype(jnp.bfloat16)
        inflight = {}
        inflight[(0, 0)] = rs_desc(0, 0)
        inflight[(0, 0)].start()
        pb = jnp.dot(h, w2_ref[:, COL:N], preferred_element_type=jnp.float32)
        acc[:, COL:N] = pb.astype(jnp.bfloat16)
        inflight[(1, 0)] = rs_desc(1, 0)
        inflight[(1, 0)].start()

        for r in range(5):
            half = RS_HALF[r]
            krows = pl.ds(rs_lo[r + 1], half)
            srows = pl.ds(RS_OFF[r], half)
            for s in (0, 1):
                cols = pl.ds(col_off[s], COL)
                inflight[(s, r)].wait()
                acc[krows, cols] = acc[krows, cols] + stage[srows, cols]
                if r < 4:
                    inflight[(s, r + 1)] = rs_desc(s, r + 1)
                    inflight[(s, r + 1)].start()

        ag = {}
        ag[(0, 0)] = ag_desc(0, 0)
        ag[(0, 0)].start()
        ag[(1, 0)] = ag_desc(1, 0)
        ag[(1, 0)].start()
        for r in range(5):
            for s in (0, 1):
                ag[(s, r)].wait()
                if r < 4:
                    ag[(s, r + 1)] = ag_desc(s, r + 1)
                    ag[(s, r + 1)].start()

        out_ref[...] = acc[...].astype(jnp.float32)

    return pl.pallas_call(
        body,
        out_shape=jax.ShapeDtypeStruct((M, N), jnp.float32),
        in_specs=[
            pl.BlockSpec(memory_space=pltpu.VMEM),
            pl.BlockSpec(memory_space=pltpu.VMEM),
            pl.BlockSpec(memory_space=pltpu.VMEM),
        ],
        out_specs=pl.BlockSpec(memory_space=pltpu.VMEM),
        scratch_shapes=[
            pltpu.VMEM((M, N), jnp.bfloat16),
            pltpu.VMEM((M, N), jnp.bfloat16),
            pltpu.SemaphoreType.DMA,
            pltpu.SemaphoreType.DMA,
            pltpu.SemaphoreType.DMA((5,)),
            pltpu.SemaphoreType.DMA((5,)),
            pltpu.SemaphoreType.DMA((5,)),
            pltpu.SemaphoreType.DMA((5,)),
        ],
        compiler_params=pltpu.CompilerParams(collective_id=0),
    )(x, W1, W2)
